# baseline (speedup 1.0000x reference)
import functools

import jax
import jax.numpy as jnp
from jax import lax
from jax.experimental import pallas as pl
from jax.experimental.pallas import tpu as pltpu
from jax.experimental.pallas import tpu_sc as plsc

_E = 64
_K = 8
_TILE = 1024
_L = 16
_NW = 32


def _matmul_body(x_ref, w_ref, lt_ref):
    x = x_ref[0]
    w = w_ref[...]
    logits = lax.dot_general(
        x, w, (((1,), (1,)), ((), ())),
        preferred_element_type=jnp.float32)
    lt_ref[...] = logits.T


def _tc_logits(x_chunk, W):
    s, d = x_chunk.shape
    nj = s // _TILE
    return pl.pallas_call(
        _matmul_body,
        grid=(1, nj),
        in_specs=[
            pl.BlockSpec((1, _TILE, d), lambda i, j: (0, j, 0)),
            pl.BlockSpec((_E, d), lambda i, j: (0, 0)),
        ],
        out_specs=pl.BlockSpec((_E, _TILE), lambda i, j: (0, j)),
        out_shape=jax.ShapeDtypeStruct((_E, s), jnp.float32),
        compiler_params=pltpu.CompilerParams(
            dimension_semantics=("arbitrary", "arbitrary"),
        ),
    )(x_chunk[None], W)


def _make_route(n):
    sub = n // _NW
    mesh = plsc.VectorSubcoreMesh(core_axis_name="c", subcore_axis_name="s")

    @functools.partial(
        pl.kernel, mesh=mesh,
        out_type=[
            jax.ShapeDtypeStruct((n * _E,), jnp.float32),
            jax.ShapeDtypeStruct((n * _K,), jnp.int32),
        ],
        scratch_types=[
            pltpu.VMEM((_E, sub), jnp.float32),
            pltpu.VMEM((sub * _E,), jnp.float32),
            pltpu.VMEM((sub * _K,), jnp.int32),
        ],
        compiler_params=pltpu.CompilerParams(needs_layout_passes=False),
    )
    def route(lt_hbm, zz_hbm, w_hbm, idx_hbm, lt_v, w_v, idx_v):
        wid = lax.axis_index("s") * 2 + lax.axis_index("c")
        base = wid * sub
        lane = lax.broadcasted_iota(jnp.int32, (_L,), 0)
        neginf = jnp.full((_L,), -jnp.inf, jnp.float32)

        pltpu.sync_copy(lt_hbm.at[:, pl.ds(base, sub)], lt_v)
        pltpu.sync_copy(zz_hbm, w_v)

        def group_body(g, c):
            work = [lt_v[e, pl.ds(g * _L, _L)] for e in range(_E)]
            ms, ams = [], []
            for j in range(_K):
                cur_v = work
                cur_i = [jnp.full((_L,), e, jnp.int32) for e in range(_E)]
                while len(cur_v) > 1:
                    nv, ni = [], []
                    for a in range(0, len(cur_v), 2):
                        take = cur_v[a + 1] > cur_v[a]
                        nv.append(jnp.maximum(cur_v[a], cur_v[a + 1]))
                        ni.append(jnp.where(take, cur_i[a + 1], cur_i[a]))
                    cur_v, cur_i = nv, ni
                m, am = cur_v[0], cur_i[0]
                ms.append(m)
                ams.append(am)
                if j + 1 < _K:
                    work = [jnp.where(am == e, neginf, work[e])
                            for e in range(_E)]
            exps = [jnp.exp(mj - ms[0]) for mj in ms]
            den = exps[0]
            for t in exps[1:]:
                den = den + t
            tok = g * _L + lane
            wbase = tok * _E
            ibase = tok * _K
            for j in range(_K):
                plsc.store_scatter(w_v, [wbase + ams[j]], exps[j] / den)
                plsc.store_scatter(idx_v, [ibase + j], ams[j])
            return c

        lax.fori_loop(0, sub // _L, group_body, 0)
        pltpu.sync_copy(w_v, w_hbm.at[pl.ds(base * _E, sub * _E)])
        pltpu.sync_copy(idx_v, idx_hbm.at[pl.ds(base * _K, sub * _K)])

    return route


def kernel(input, W):
    b, s, d = input.shape
    route = _make_route(s)
    zz = jnp.zeros((s // _NW * _E,), jnp.float32)
    ws, idxs = [], []
    for c in range(b):
        lt = _tc_logits(input[c], W)
        w_c, idx_c = route(lt, zz)
        ws.append(w_c.reshape(s, _E))
        idxs.append(idx_c.reshape(s, _K))
    return (jnp.stack(ws), jnp.stack(idxs))


# speedup vs baseline: 3.1443x; 3.1443x over previous
import functools

import jax
import jax.numpy as jnp
from jax.experimental import pallas as pl
from jax.experimental.pallas import tpu as pltpu

_NUM_EXPERTS = 64
_TOP_K = 8
_TILE = 1024


def _router_body(x_ref, w_ref, w_out_ref, idx_out_ref):
    x = x_ref[0]
    w = w_ref[...]
    logits = jax.lax.dot_general(
        x, w, (((1,), (1,)), ((), ())),
        preferred_element_type=jnp.float32)
    lt = logits.T
    t = lt.shape[1]
    row = jax.lax.broadcasted_iota(jnp.int32, lt.shape, 0)
    row8 = jax.lax.broadcasted_iota(jnp.int32, (_TOP_K, t), 0)
    work = lt
    sel = jnp.zeros(lt.shape, dtype=jnp.bool_)
    idx_t = jnp.zeros((_TOP_K, t), jnp.int32)
    max0 = None
    for j in range(_TOP_K):
        m = jnp.max(work, axis=0, keepdims=True)
        if j == 0:
            max0 = m
        amax = jnp.min(jnp.where(work == m, row, _NUM_EXPERTS),
                       axis=0, keepdims=True)
        hit = row == amax
        sel = jnp.logical_or(sel, hit)
        work = jnp.where(hit, -jnp.inf, work)
        idx_t = jnp.where(row8 == j, amax, idx_t)
    e = jnp.where(sel, jnp.exp(lt - max0), 0.0)
    denom = jnp.sum(e, axis=0, keepdims=True)
    w_out_ref[0] = (e / denom).T
    idx_out_ref[0] = idx_t.T


def kernel(input, W):
    b, s, d = input.shape
    e = W.shape[0]
    tile = _TILE
    grid = (b, s // tile)
    weights, idx = pl.pallas_call(
        _router_body,
        grid=grid,
        in_specs=[
            pl.BlockSpec((1, tile, d), lambda i, j: (i, j, 0)),
            pl.BlockSpec((e, d), lambda i, j: (0, 0)),
        ],
        out_specs=[
            pl.BlockSpec((1, tile, e), lambda i, j: (i, j, 0)),
            pl.BlockSpec((1, tile, _TOP_K), lambda i, j: (i, j, 0)),
        ],
        out_shape=[
            jax.ShapeDtypeStruct((b, s, e), jnp.float32),
            jax.ShapeDtypeStruct((b, s, _TOP_K), jnp.int32),
        ],
        compiler_params=pltpu.CompilerParams(
            dimension_semantics=("arbitrary", "arbitrary"),
        ),
    )(input, W)
    return weights, idx


# speedup vs baseline: 3.1688x; 1.0078x over previous
import functools

import jax
import jax.numpy as jnp
from jax.experimental import pallas as pl
from jax.experimental.pallas import tpu as pltpu

_NUM_EXPERTS = 64
_TOP_K = 8
_TILE = 1024


def _router_body(x_ref, w_ref, w_out_ref, idx_out_ref):
    x = x_ref[0]
    w = w_ref[...]
    logits = jax.lax.dot_general(
        x, w, (((1,), (1,)), ((), ())),
        preferred_element_type=jnp.float32)
    lt = logits.T
    t = lt.shape[1]
    row = jax.lax.broadcasted_iota(jnp.int32, lt.shape, 0)
    row8 = jax.lax.broadcasted_iota(jnp.int32, (_TOP_K, t), 0)
    work = lt
    sel = jnp.zeros(lt.shape, dtype=jnp.bool_)
    idx_t = jnp.zeros((_TOP_K, t), jnp.int32)
    max0 = None
    for j in range(_TOP_K):
        m = jnp.max(work, axis=0, keepdims=True)
        if j == 0:
            max0 = m
        amax = jnp.min(jnp.where(work == m, row, _NUM_EXPERTS),
                       axis=0, keepdims=True)
        hit = row == amax
        sel = jnp.logical_or(sel, hit)
        work = jnp.where(hit, -jnp.inf, work)
        idx_t = jnp.where(row8 == j, amax, idx_t)
    e = jnp.where(sel, jnp.exp(lt - max0), 0.0)
    denom = jnp.sum(e, axis=0, keepdims=True)
    w_out_ref[0] = (e / denom).T
    idx_out_ref[0] = idx_t.T


def kernel(input, W):
    b, s, d = input.shape
    e = W.shape[0]
    tile = _TILE
    grid = (b, s // tile)
    weights, idx = pl.pallas_call(
        _router_body,
        grid=grid,
        in_specs=[
            pl.BlockSpec((1, tile, d), lambda i, j: (i, j, 0)),
            pl.BlockSpec((e, d), lambda i, j: (0, 0)),
        ],
        out_specs=[
            pl.BlockSpec((1, tile, e), lambda i, j: (i, j, 0)),
            pl.BlockSpec((1, tile, _TOP_K), lambda i, j: (i, j, 0)),
        ],
        out_shape=[
            jax.ShapeDtypeStruct((b, s, e), jnp.float32),
            jax.ShapeDtypeStruct((b, s, _TOP_K), jnp.int32),
        ],
        compiler_params=pltpu.CompilerParams(
            dimension_semantics=("parallel", "parallel"),
        ),
    )(input, W)
    return weights, idx
